# baseline (speedup 1.0000x reference)
import functools

import jax
import jax.numpy as jnp
from jax import lax
from jax.experimental import pallas as pl
from jax.experimental.pallas import tpu as pltpu
from jax.experimental.pallas import tpu_sc as plsc

D = 256
K = 8192
T = 4096
L = 16
NC = 2
NS = 16
NW = NC * NS
DC = D // L
RW = T // NW
SW = K // NW
UW = K // NW
PCH = 64
SCH = 128

_mesh = plsc.VectorSubcoreMesh(
    core_axis_name="c", subcore_axis_name="s", num_cores=NC, num_subcores=NS
)


def _wid():
    return lax.axis_index("s") * NC + lax.axis_index("c")


def _tree_hsum(accs, lane):
    idx_e = (lane % 8) * 2
    idx_o = idx_e + 1
    lo = lane < 8

    def combine(a, b):
        a_e = a.at[idx_e].get(mode="promise_in_bounds")
        a_o = a.at[idx_o].get(mode="promise_in_bounds")
        b_e = b.at[idx_e].get(mode="promise_in_bounds")
        b_o = b.at[idx_o].get(mode="promise_in_bounds")
        return jnp.where(lo, a_e + a_o, b_e + b_o)

    level = accs
    while len(level) > 1:
        level = [combine(level[2 * k], level[2 * k + 1])
                 for k in range(len(level) // 2)]
    return level[0]


@functools.partial(
    pl.kernel,
    out_type=jax.ShapeDtypeStruct((NW, D), jnp.float32),
    mesh=_mesh,
    scratch_types=[
        pltpu.VMEM((PCH, D), jnp.float32),
        pltpu.VMEM((PCH, D), jnp.float32),
        pltpu.VMEM((D,), jnp.float32),
        pltpu.SemaphoreType.DMA,
        pltpu.SemaphoreType.DMA,
    ],
)
def _pool(hid_hbm, out_hbm, buf0, buf1, qrow, sem0, sem1):
    w = _wid()
    r0 = w * RW
    cp0 = pltpu.make_async_copy(hid_hbm.at[pl.ds(r0, PCH)], buf0, sem0)
    cp0.start()
    cp1 = pltpu.make_async_copy(hid_hbm.at[pl.ds(r0 + PCH, PCH)], buf1, sem1)
    cp1.start()
    zeros = jnp.zeros((L,), jnp.float32)

    def accum(buf, accs):
        @plsc.parallel_loop(0, PCH, carry=accs)
        def final(r, a):
            return tuple(a[cc] + buf[r, pl.ds(cc * L, L)] for cc in range(DC))
        return final

    cp0.wait()
    accs = accum(buf0, (zeros,) * DC)
    cp1.wait()
    accs = accum(buf1, accs)
    for cc in range(DC):
        qrow[pl.ds(cc * L, L)] = accs[cc]
    pltpu.sync_copy(qrow, out_hbm.at[w])


@functools.partial(
    pl.kernel,
    out_type=jax.ShapeDtypeStruct((NW, L), jnp.float32),
    mesh=_mesh,
    scratch_types=[
        pltpu.VMEM((SCH, D), jnp.float32),
        pltpu.VMEM((SCH, D), jnp.float32),
        pltpu.VMEM((NW, D), jnp.float32),
        pltpu.VMEM((L,), jnp.float32),
        pltpu.SemaphoreType.DMA,
        pltpu.SemaphoreType.DMA,
    ],
)
def _dist(seed_hbm, part_hbm, cand_hbm, buf0, buf1, ptmp, crow, sem0, sem1):
    w = _wid()
    lane = lax.iota(jnp.int32, L)
    sbase = w * SW
    cp0 = pltpu.make_async_copy(seed_hbm.at[pl.ds(sbase, SCH)], buf0, sem0)
    cp0.start()
    cp1 = pltpu.make_async_copy(seed_hbm.at[pl.ds(sbase + SCH, SCH)], buf1, sem1)
    cp1.start()

    pltpu.sync_copy(part_hbm, ptmp)
    q = []
    inv_t = 1.0 / T
    for cc in range(DC):
        acc = ptmp[0, pl.ds(cc * L, L)]
        for r in range(1, NW):
            acc = acc + ptmp[r, pl.ds(cc * L, L)]
        q.append(acc * inv_t)

    best_d = jnp.full((L,), jnp.inf, jnp.float32)
    best_i = jnp.zeros((L,), jnp.int32)

    def process_chunk(buf, base, bd0, bi0):
        @plsc.parallel_loop(0, SCH // L, carry=(bd0, bi0))
        def final(b, carry):
            bd, bi = carry
            accs = []
            for j in range(L):
                row = b * L + j
                acc = None
                for cc in range(DC):
                    dv = buf[row, pl.ds(cc * L, L)] - q[cc]
                    acc = dv * dv if acc is None else acc + dv * dv
                accs.append(acc)
            dist = _tree_hsum(accs, lane)
            idx = base + b * L + lane
            better = dist < bd
            return jnp.where(better, dist, bd), jnp.where(better, idx, bi)
        return final

    cp0.wait()
    cp1.wait()
    best_d, best_i = process_chunk(buf0, sbase, best_d, best_i)
    best_d, best_i = process_chunk(buf1, sbase + SCH, best_d, best_i)

    d_best = best_d[0]
    i_best = best_i[0]
    for l in range(1, L):
        dl = best_d[l]
        il = best_i[l]
        better = dl < d_best
        d_best = lax.select(better, dl, d_best)
        i_best = lax.select(better, il, i_best)
    crow[...] = jnp.where(lane == 0, d_best,
                          jnp.where(lane == 1, i_best.astype(jnp.float32), 0.0))
    pltpu.sync_copy(crow, cand_hbm.at[w])


@functools.partial(
    pl.kernel,
    out_type=(
        jax.ShapeDtypeStruct((L,), jnp.int32),
        jax.ShapeDtypeStruct((K,), jnp.float32),
    ),
    mesh=_mesh,
    scratch_types=[
        pltpu.VMEM((NW, L), jnp.float32),
        pltpu.VMEM((UW,), jnp.float32),
        pltpu.VMEM((L,), jnp.int32),
    ],
)
def _merge(cand_hbm, usage_hbm, idx_hbm, usage_out_hbm, ctmp, usv, idxv):
    w = _wid()
    lane = lax.iota(jnp.int32, L)
    pltpu.sync_copy(cand_hbm, ctmp)

    def merge_step(i, carry):
        gd, gi = carry
        v = ctmp[i, pl.ds(0, L)]
        d = v[0]
        ind = v[1]
        better = d < gd
        return (lax.select(better, d, gd), lax.select(better, ind, gi))

    _, gi_f = lax.fori_loop(0, NW, merge_step,
                            (jnp.float32(jnp.inf), jnp.float32(0.0)))
    winner = gi_f.astype(jnp.int32)

    ubase = w * UW
    pltpu.sync_copy(usage_hbm.at[pl.ds(ubase, UW)], usv)
    off = winner - ubase

    @pl.when((off >= 0) & (off < UW))
    def _():
        blk = (off // L) * L
        vec = usv[pl.ds(pl.multiple_of(blk, L), L)]
        usv[pl.ds(pl.multiple_of(blk, L), L)] = vec + jnp.where(
            lane == off - blk, 1.0, 0.0)

    pltpu.sync_copy(usv, usage_out_hbm.at[pl.ds(ubase, UW)])

    @pl.when(w == 0)
    def _():
        idxv[...] = jnp.full((L,), winner, jnp.int32)
        pltpu.sync_copy(idxv, idx_hbm)


@jax.jit
def kernel(hidden_state, seed_bank, usage_frequency):
    partials = _pool(hidden_state)
    cand = _dist(seed_bank, partials)
    idx16, usage_new = _merge(cand, usage_frequency)
    return idx16[:1], usage_new


# speedup vs baseline: 1.0023x; 1.0023x over previous
import functools

import jax
import jax.numpy as jnp
from jax import lax
from jax.experimental import pallas as pl
from jax.experimental.pallas import tpu as pltpu
from jax.experimental.pallas import tpu_sc as plsc

D = 256
K = 8192
T = 4096
L = 16
NC = 2
NS = 16
NW = NC * NS
DC = D // L
RW = T // NW
SW = K // NW
UW = K // NW
PCH = 64
SCH = 128

_mesh = plsc.VectorSubcoreMesh(
    core_axis_name="c", subcore_axis_name="s", num_cores=NC, num_subcores=NS
)


def _wid():
    return lax.axis_index("s") * NC + lax.axis_index("c")


def _tree_hsum(accs, lane):
    idx_e = (lane % 8) * 2
    idx_o = idx_e + 1
    lo = lane < 8

    def combine(a, b):
        a_e = a.at[idx_e].get(mode="promise_in_bounds")
        a_o = a.at[idx_o].get(mode="promise_in_bounds")
        b_e = b.at[idx_e].get(mode="promise_in_bounds")
        b_o = b.at[idx_o].get(mode="promise_in_bounds")
        return jnp.where(lo, a_e + a_o, b_e + b_o)

    level = accs
    while len(level) > 1:
        level = [combine(level[2 * k], level[2 * k + 1])
                 for k in range(len(level) // 2)]
    return level[0]


@functools.partial(
    pl.kernel,
    out_type=jax.ShapeDtypeStruct((NW, D), jnp.float32),
    mesh=_mesh,
    scratch_types=[
        pltpu.VMEM((PCH, D), jnp.float32),
        pltpu.VMEM((PCH, D), jnp.float32),
        pltpu.VMEM((D,), jnp.float32),
        pltpu.SemaphoreType.DMA,
        pltpu.SemaphoreType.DMA,
    ],
)
def _pool(hid_hbm, out_hbm, buf0, buf1, qrow, sem0, sem1):
    w = _wid()
    r0 = w * RW
    cp0 = pltpu.make_async_copy(hid_hbm.at[pl.ds(r0, PCH)], buf0, sem0)
    cp0.start()
    cp1 = pltpu.make_async_copy(hid_hbm.at[pl.ds(r0 + PCH, PCH)], buf1, sem1)
    cp1.start()
    zeros = jnp.zeros((L,), jnp.float32)

    def accum(buf, accs):
        @plsc.parallel_loop(0, PCH, carry=accs)
        def final(r, a):
            return tuple(a[cc] + buf[r, pl.ds(cc * L, L)] for cc in range(DC))
        return final

    cp0.wait()
    accs = accum(buf0, (zeros,) * DC)
    cp1.wait()
    accs = accum(buf1, accs)
    for cc in range(DC):
        qrow[pl.ds(cc * L, L)] = accs[cc]
    pltpu.sync_copy(qrow, out_hbm.at[w])


@functools.partial(
    pl.kernel,
    out_type=jax.ShapeDtypeStruct((NW, L), jnp.float32),
    mesh=_mesh,
    scratch_types=[
        pltpu.VMEM((SCH, D), jnp.float32),
        pltpu.VMEM((SCH, D), jnp.float32),
        pltpu.VMEM((NW, D), jnp.float32),
        pltpu.VMEM((L,), jnp.float32),
        pltpu.SemaphoreType.DMA,
        pltpu.SemaphoreType.DMA,
    ],
)
def _dist(seed_hbm, part_hbm, cand_hbm, buf0, buf1, ptmp, crow, sem0, sem1):
    w = _wid()
    lane = lax.iota(jnp.int32, L)
    sbase = w * SW
    cp0 = pltpu.make_async_copy(seed_hbm.at[pl.ds(sbase, SCH)], buf0, sem0)
    cp0.start()
    cp1 = pltpu.make_async_copy(seed_hbm.at[pl.ds(sbase + SCH, SCH)], buf1, sem1)
    cp1.start()

    pltpu.sync_copy(part_hbm, ptmp)
    q = []
    inv_t = 1.0 / T
    for cc in range(DC):
        acc = ptmp[0, pl.ds(cc * L, L)]
        for r in range(1, NW):
            acc = acc + ptmp[r, pl.ds(cc * L, L)]
        q.append(acc * inv_t)

    best_d = jnp.full((L,), jnp.inf, jnp.float32)
    best_i = jnp.zeros((L,), jnp.int32)

    def process_chunk(buf, base, bd0, bi0):
        @plsc.parallel_loop(0, SCH // L, carry=(bd0, bi0), unroll=2)
        def final(b, carry):
            bd, bi = carry
            accs = []
            for j in range(L):
                row = b * L + j
                acc = None
                for cc in range(DC):
                    dv = buf[row, pl.ds(cc * L, L)] - q[cc]
                    acc = dv * dv if acc is None else acc + dv * dv
                accs.append(acc)
            dist = _tree_hsum(accs, lane)
            idx = base + b * L + lane
            better = dist < bd
            return jnp.where(better, dist, bd), jnp.where(better, idx, bi)
        return final

    cp0.wait()
    cp1.wait()
    best_d, best_i = process_chunk(buf0, sbase, best_d, best_i)
    best_d, best_i = process_chunk(buf1, sbase + SCH, best_d, best_i)

    d_best = best_d[0]
    i_best = best_i[0]
    for l in range(1, L):
        dl = best_d[l]
        il = best_i[l]
        better = dl < d_best
        d_best = lax.select(better, dl, d_best)
        i_best = lax.select(better, il, i_best)
    crow[...] = jnp.where(lane == 0, d_best,
                          jnp.where(lane == 1, i_best.astype(jnp.float32), 0.0))
    pltpu.sync_copy(crow, cand_hbm.at[w])


@functools.partial(
    pl.kernel,
    out_type=(
        jax.ShapeDtypeStruct((L,), jnp.int32),
        jax.ShapeDtypeStruct((K,), jnp.float32),
    ),
    mesh=_mesh,
    scratch_types=[
        pltpu.VMEM((NW, L), jnp.float32),
        pltpu.VMEM((UW,), jnp.float32),
        pltpu.VMEM((L,), jnp.int32),
    ],
)
def _merge(cand_hbm, usage_hbm, idx_hbm, usage_out_hbm, ctmp, usv, idxv):
    w = _wid()
    lane = lax.iota(jnp.int32, L)
    pltpu.sync_copy(cand_hbm, ctmp)

    def merge_step(i, carry):
        gd, gi = carry
        v = ctmp[i, pl.ds(0, L)]
        d = v[0]
        ind = v[1]
        better = d < gd
        return (lax.select(better, d, gd), lax.select(better, ind, gi))

    _, gi_f = lax.fori_loop(0, NW, merge_step,
                            (jnp.float32(jnp.inf), jnp.float32(0.0)))
    winner = gi_f.astype(jnp.int32)

    ubase = w * UW
    pltpu.sync_copy(usage_hbm.at[pl.ds(ubase, UW)], usv)
    off = winner - ubase

    @pl.when((off >= 0) & (off < UW))
    def _():
        blk = (off // L) * L
        vec = usv[pl.ds(pl.multiple_of(blk, L), L)]
        usv[pl.ds(pl.multiple_of(blk, L), L)] = vec + jnp.where(
            lane == off - blk, 1.0, 0.0)

    pltpu.sync_copy(usv, usage_out_hbm.at[pl.ds(ubase, UW)])

    @pl.when(w == 0)
    def _():
        idxv[...] = jnp.full((L,), winner, jnp.int32)
        pltpu.sync_copy(idxv, idx_hbm)


@jax.jit
def kernel(hidden_state, seed_bank, usage_frequency):
    partials = _pool(hidden_state)
    cand = _dist(seed_bank, partials)
    idx16, usage_new = _merge(cand, usage_frequency)
    return idx16[:1], usage_new


# speedup vs baseline: 1.1058x; 1.1033x over previous
import functools

import jax
import jax.numpy as jnp
from jax import lax
from jax.experimental import pallas as pl
from jax.experimental.pallas import tpu as pltpu
from jax.experimental.pallas import tpu_sc as plsc

D = 256
K = 8192
T = 4096
L = 16
NC = 2
NS = 16
NW = NC * NS
DC = D // L
RW = T // NW
SW = K // NW
UW = K // NW
PCH = 64
SCH = 128

_mesh = plsc.VectorSubcoreMesh(
    core_axis_name="c", subcore_axis_name="s", num_cores=NC, num_subcores=NS
)


def _wid():
    return lax.axis_index("s") * NC + lax.axis_index("c")


def _tree_hsum(accs, lane):
    idx_e = (lane % 8) * 2
    idx_o = idx_e + 1
    lo = lane < 8

    def combine(a, b):
        a_e = a.at[idx_e].get(mode="promise_in_bounds")
        a_o = a.at[idx_o].get(mode="promise_in_bounds")
        b_e = b.at[idx_e].get(mode="promise_in_bounds")
        b_o = b.at[idx_o].get(mode="promise_in_bounds")
        return jnp.where(lo, a_e + a_o, b_e + b_o)

    level = accs
    while len(level) > 1:
        level = [combine(level[2 * k], level[2 * k + 1])
                 for k in range(len(level) // 2)]
    return level[0]


@functools.partial(
    pl.kernel,
    out_type=jax.ShapeDtypeStruct((NW, D), jnp.float32),
    mesh=_mesh,
    scratch_types=[
        pltpu.VMEM((PCH, D), jnp.float32),
        pltpu.VMEM((PCH, D), jnp.float32),
        pltpu.VMEM((D,), jnp.float32),
        pltpu.SemaphoreType.DMA,
        pltpu.SemaphoreType.DMA,
    ],
)
def _pool(hid_hbm, out_hbm, buf0, buf1, qrow, sem0, sem1):
    w = _wid()
    r0 = w * RW
    cp0 = pltpu.make_async_copy(hid_hbm.at[pl.ds(r0, PCH)], buf0, sem0)
    cp0.start()
    cp1 = pltpu.make_async_copy(hid_hbm.at[pl.ds(r0 + PCH, PCH)], buf1, sem1)
    cp1.start()
    zeros = jnp.zeros((L,), jnp.float32)

    def accum(buf, accs):
        @plsc.parallel_loop(0, PCH, carry=accs)
        def final(r, a):
            return tuple(a[cc] + buf[r, pl.ds(cc * L, L)] for cc in range(DC))
        return final

    cp0.wait()
    accs = accum(buf0, (zeros,) * DC)
    cp1.wait()
    accs = accum(buf1, accs)
    for cc in range(DC):
        qrow[pl.ds(cc * L, L)] = accs[cc]
    pltpu.sync_copy(qrow, out_hbm.at[w])


@functools.partial(
    pl.kernel,
    out_type=jax.ShapeDtypeStruct((NW, L), jnp.float32),
    mesh=_mesh,
    scratch_types=[
        pltpu.VMEM((SCH, D), jnp.float32),
        pltpu.VMEM((SCH, D), jnp.float32),
        pltpu.VMEM((NW, D), jnp.float32),
        pltpu.VMEM((L,), jnp.float32),
        pltpu.SemaphoreType.DMA,
        pltpu.SemaphoreType.DMA,
    ],
)
def _dist(seed_hbm, part_hbm, cand_hbm, buf0, buf1, ptmp, crow, sem0, sem1):
    w = _wid()
    lane = lax.iota(jnp.int32, L)
    sbase = w * SW
    cp0 = pltpu.make_async_copy(seed_hbm.at[pl.ds(sbase, SCH)], buf0, sem0)
    cp0.start()
    cp1 = pltpu.make_async_copy(seed_hbm.at[pl.ds(sbase + SCH, SCH)], buf1, sem1)
    cp1.start()

    pltpu.sync_copy(part_hbm, ptmp)
    q = []
    inv_t = 1.0 / T
    for cc in range(DC):
        acc = ptmp[0, pl.ds(cc * L, L)]
        for r in range(1, NW):
            acc = acc + ptmp[r, pl.ds(cc * L, L)]
        q.append(acc * inv_t)

    best_d = jnp.full((L,), jnp.inf, jnp.float32)
    best_i = jnp.zeros((L,), jnp.int32)

    def process_chunk(buf, base, bd0, bi0):
        @plsc.parallel_loop(0, SCH // L, carry=(bd0, bi0), unroll=2)
        def final(b, carry):
            bd, bi = carry
            accs = []
            for j in range(L):
                row = b * L + j
                acc = None
                for cc in range(DC):
                    dv = buf[row, pl.ds(cc * L, L)] - q[cc]
                    acc = dv * dv if acc is None else acc + dv * dv
                accs.append(acc)
            dist = _tree_hsum(accs, lane)
            idx = base + b * L + lane
            better = dist < bd
            return jnp.where(better, dist, bd), jnp.where(better, idx, bi)
        return final

    cp0.wait()
    cp1.wait()
    best_d, best_i = process_chunk(buf0, sbase, best_d, best_i)
    best_d, best_i = process_chunk(buf1, sbase + SCH, best_d, best_i)

    d_best = best_d[0]
    i_best = best_i[0]
    for l in range(1, L):
        dl = best_d[l]
        il = best_i[l]
        better = dl < d_best
        d_best = lax.select(better, dl, d_best)
        i_best = lax.select(better, il, i_best)
    crow[...] = jnp.where(lane == 0, d_best,
                          jnp.where(lane == 1, i_best.astype(jnp.float32), 0.0))
    pltpu.sync_copy(crow, cand_hbm.at[w])


def _finish_body(cand_ref, usage_ref, idx_ref, out_ref):
    cand = cand_ref[...]
    rows = lax.broadcasted_iota(jnp.int32, (NW, L), 0)
    cols = lax.broadcasted_iota(jnp.int32, (NW, L), 1)
    dmat = jnp.where(cols == 0, cand, jnp.inf)
    dmin = jnp.min(dmat)
    win_row = jnp.min(jnp.where(dmat == dmin, rows, jnp.int32(2 ** 30)))
    winner = jnp.sum(
        jnp.where((cols == 1) & (rows == win_row), cand, 0.0)
    ).astype(jnp.int32)
    r64 = lax.broadcasted_iota(jnp.int32, (K // 128, 128), 0)
    c128 = lax.broadcasted_iota(jnp.int32, (K // 128, 128), 1)
    lin = r64 * 128 + c128
    out_ref[...] = usage_ref[...] + jnp.where(lin == winner, 1.0, 0.0)
    idx_ref[...] = jnp.full((1, 1), winner, jnp.int32)


_finish = pl.pallas_call(
    _finish_body,
    out_shape=(
        jax.ShapeDtypeStruct((1, 1), jnp.int32),
        jax.ShapeDtypeStruct((K // 128, 128), jnp.float32),
    ),
)


@jax.jit
def kernel(hidden_state, seed_bank, usage_frequency):
    partials = _pool(hidden_state)
    cand = _dist(seed_bank, partials)
    idx11, usage2 = _finish(cand, usage_frequency.reshape(K // 128, 128))
    return idx11.reshape(1), usage2.reshape(K)


# speedup vs baseline: 1.4616x; 1.3217x over previous
import functools

import jax
import jax.numpy as jnp
from jax import lax
from jax.experimental import pallas as pl
from jax.experimental.pallas import tpu as pltpu
from jax.experimental.pallas import tpu_sc as plsc

D = 256
K = 8192
T = 4096
L = 16
NC = 2
NS = 16
NW = NC * NS
DC = D // L
RW = T // NW
SW = K // NW
UW = K // NW
PCH = 64
SCH = 128

_mesh = plsc.VectorSubcoreMesh(
    core_axis_name="c", subcore_axis_name="s", num_cores=NC, num_subcores=NS
)


def _wid():
    return lax.axis_index("s") * NC + lax.axis_index("c")


def _tree_hsum(accs, lane):
    idx_e = (lane % 8) * 2
    idx_o = idx_e + 1
    lo = lane < 8

    def combine(a, b):
        a_e = a.at[idx_e].get(mode="promise_in_bounds")
        a_o = a.at[idx_o].get(mode="promise_in_bounds")
        b_e = b.at[idx_e].get(mode="promise_in_bounds")
        b_o = b.at[idx_o].get(mode="promise_in_bounds")
        return jnp.where(lo, a_e + a_o, b_e + b_o)

    level = accs
    while len(level) > 1:
        level = [combine(level[2 * k], level[2 * k + 1])
                 for k in range(len(level) // 2)]
    return level[0]


def _pool_body(hid_ref, out_ref):
    out_ref[...] = jnp.sum(hid_ref[...], axis=0, keepdims=True) * (1.0 / T)


_pool = pl.pallas_call(
    _pool_body,
    out_shape=jax.ShapeDtypeStruct((1, D), jnp.float32),
)


@functools.partial(
    pl.kernel,
    out_type=jax.ShapeDtypeStruct((NW, L), jnp.float32),
    mesh=_mesh,
    scratch_types=[
        pltpu.VMEM((SCH, D), jnp.float32),
        pltpu.VMEM((SCH, D), jnp.float32),
        pltpu.VMEM((1, D), jnp.float32),
        pltpu.VMEM((L,), jnp.float32),
        pltpu.SemaphoreType.DMA,
        pltpu.SemaphoreType.DMA,
    ],
)
def _dist(seed_hbm, pool_hbm, cand_hbm, buf0, buf1, ptmp, crow, sem0, sem1):
    w = _wid()
    lane = lax.iota(jnp.int32, L)
    sbase = w * SW
    cp0 = pltpu.make_async_copy(seed_hbm.at[pl.ds(sbase, SCH)], buf0, sem0)
    cp0.start()
    cp1 = pltpu.make_async_copy(seed_hbm.at[pl.ds(sbase + SCH, SCH)], buf1, sem1)
    cp1.start()

    pltpu.sync_copy(pool_hbm, ptmp)
    q = [ptmp[0, pl.ds(cc * L, L)] for cc in range(DC)]

    best_d = jnp.full((L,), jnp.inf, jnp.float32)
    best_i = jnp.zeros((L,), jnp.int32)

    def process_chunk(buf, base, bd0, bi0):
        @plsc.parallel_loop(0, SCH // L, carry=(bd0, bi0), unroll=2)
        def final(b, carry):
            bd, bi = carry
            accs = []
            for j in range(L):
                row = b * L + j
                acc = None
                for cc in range(DC):
                    dv = buf[row, pl.ds(cc * L, L)] - q[cc]
                    acc = dv * dv if acc is None else acc + dv * dv
                accs.append(acc)
            dist = _tree_hsum(accs, lane)
            idx = base + b * L + lane
            better = dist < bd
            return jnp.where(better, dist, bd), jnp.where(better, idx, bi)
        return final

    cp0.wait()
    cp1.wait()
    best_d, best_i = process_chunk(buf0, sbase, best_d, best_i)
    best_d, best_i = process_chunk(buf1, sbase + SCH, best_d, best_i)

    d_best = best_d[0]
    i_best = best_i[0]
    for l in range(1, L):
        dl = best_d[l]
        il = best_i[l]
        better = dl < d_best
        d_best = lax.select(better, dl, d_best)
        i_best = lax.select(better, il, i_best)
    crow[...] = jnp.where(lane == 0, d_best,
                          jnp.where(lane == 1, i_best.astype(jnp.float32), 0.0))
    pltpu.sync_copy(crow, cand_hbm.at[w])


def _finish_body(cand_ref, usage_ref, idx_ref, out_ref):
    cand = cand_ref[...]
    rows = lax.broadcasted_iota(jnp.int32, (NW, L), 0)
    cols = lax.broadcasted_iota(jnp.int32, (NW, L), 1)
    dmat = jnp.where(cols == 0, cand, jnp.inf)
    dmin = jnp.min(dmat)
    win_row = jnp.min(jnp.where(dmat == dmin, rows, jnp.int32(2 ** 30)))
    winner = jnp.sum(
        jnp.where((cols == 1) & (rows == win_row), cand, 0.0)
    ).astype(jnp.int32)
    r64 = lax.broadcasted_iota(jnp.int32, (K // 128, 128), 0)
    c128 = lax.broadcasted_iota(jnp.int32, (K // 128, 128), 1)
    lin = r64 * 128 + c128
    out_ref[...] = usage_ref[...] + jnp.where(lin == winner, 1.0, 0.0)
    idx_ref[...] = jnp.full((1, 1), winner, jnp.int32)


_finish = pl.pallas_call(
    _finish_body,
    out_shape=(
        jax.ShapeDtypeStruct((1, 1), jnp.int32),
        jax.ShapeDtypeStruct((K // 128, 128), jnp.float32),
    ),
)


@jax.jit
def kernel(hidden_state, seed_bank, usage_frequency):
    pooled = _pool(hidden_state)
    cand = _dist(seed_bank, pooled)
    idx11, usage2 = _finish(cand, usage_frequency.reshape(K // 128, 128))
    return idx11.reshape(1), usage2.reshape(K)
